# Initial kernel scaffold; baseline (speedup 1.0000x reference)
import jax
import jax.numpy as jnp
from jax.experimental import pallas as pl


def kernel(x, table):
    raise NotImplementedError("write your pallas kernel here")


# speedup vs baseline: 1.5752x; 1.5752x over previous
import functools

import jax
import jax.numpy as jnp
from jax import lax
from jax.experimental import pallas as pl
from jax.experimental.pallas import tpu as pltpu
from jax.experimental.pallas import tpu_sc as plsc

_BATCH = 16384
_HIST = 50
_DIM = 64
_B = _BATCH * _HIST
_NC = 2
_NS = 16
_NW = _NC * _NS
_B_PER_W = _B // _NW
_C = 128
_NCHUNKS = _B_PER_W // _C


def _gather_body(table_hbm, idx_hbm, out_hbm, idx_v, rows_v, sem):
    wid = lax.axis_index("s") * _NC + lax.axis_index("c")
    base = wid * _B_PER_W

    def step(i, carry):
        off = base + i * _C
        pltpu.sync_copy(idx_hbm.at[pl.ds(off, _C)], idx_v)
        pltpu.async_copy(table_hbm.at[idx_v], rows_v, sem).wait()
        pltpu.sync_copy(rows_v, out_hbm.at[pl.ds(off, _C)])
        return carry

    lax.fori_loop(0, _NCHUNKS, step, 0)


@jax.jit
def kernel(x, table):
    idx = x.reshape(_B)
    mesh = plsc.VectorSubcoreMesh(core_axis_name="c", subcore_axis_name="s")
    out = pl.kernel(
        _gather_body,
        mesh=mesh,
        out_type=jax.ShapeDtypeStruct((_B, _DIM), jnp.float32),
        scratch_types=[
            pltpu.VMEM((_C,), jnp.int32),
            pltpu.VMEM((_C, _DIM), jnp.float32),
            pltpu.SemaphoreType.DMA,
        ],
        compiler_params=pltpu.CompilerParams(use_tc_tiling_on_sc=False),
    )(table, idx)
    return out.reshape(_BATCH, _HIST, _DIM)


# speedup vs baseline: 1.8743x; 1.1899x over previous
import jax
import jax.numpy as jnp
from jax import lax
from jax.experimental import pallas as pl
from jax.experimental.pallas import tpu as pltpu
from jax.experimental.pallas import tpu_sc as plsc

_BATCH = 16384
_HIST = 50
_DIM = 64
_B = _BATCH * _HIST
_NC = 2
_NS = 16
_NW = _NC * _NS
_B_PER_W = _B // _NW
_C = 128
_NCHUNKS = _B_PER_W // _C
_NBUF = 8
_NGROUPS = _NCHUNKS // _NBUF


def _gather_body(table_hbm, idx_hbm, out_hbm, idx_all, rows, *sems):
    gsem = sems[:_NBUF]
    ssem = sems[_NBUF:]
    wid = lax.axis_index("s") * _NC + lax.axis_index("c")
    base = wid * _B_PER_W

    pltpu.sync_copy(idx_hbm.at[wid], idx_all)

    def start_gather(slot, chunk):
        pltpu.async_copy(table_hbm.at[idx_all.at[chunk]], rows.at[slot],
                         gsem[slot])

    def wait_gather(slot):
        pltpu.make_async_copy(table_hbm.at[idx_all.at[0]], rows.at[slot],
                              gsem[slot]).wait()

    def start_store(slot, chunk):
        pltpu.async_copy(rows.at[slot],
                         out_hbm.at[pl.ds(base + chunk * _C, _C)],
                         ssem[slot])

    def wait_store(slot):
        pltpu.make_async_copy(rows.at[slot], out_hbm.at[pl.ds(base, _C)],
                              ssem[slot]).wait()

    for b in range(_NBUF):
        start_gather(b, b)

    def group(g, carry):
        for b in range(_NBUF):
            i = g * _NBUF + b
            wait_gather(b)
            start_store(b, i)
            wait_store(b)
            start_gather(b, i + _NBUF)
        return carry

    lax.fori_loop(0, _NGROUPS - 1, group, 0)

    for b in range(_NBUF):
        wait_gather(b)
        start_store(b, (_NGROUPS - 1) * _NBUF + b)
    for b in range(_NBUF):
        wait_store(b)


@jax.jit
def kernel(x, table):
    idx = x.reshape(_NW, _NCHUNKS, _C)
    mesh = plsc.VectorSubcoreMesh(core_axis_name="c", subcore_axis_name="s")
    out = pl.kernel(
        _gather_body,
        mesh=mesh,
        out_type=jax.ShapeDtypeStruct((_B, _DIM), jnp.float32),
        scratch_types=[
            pltpu.VMEM((_NCHUNKS, _C), jnp.int32),
            pltpu.VMEM((_NBUF, _C, _DIM), jnp.float32),
        ] + [pltpu.SemaphoreType.DMA] * (2 * _NBUF),
        compiler_params=pltpu.CompilerParams(use_tc_tiling_on_sc=False),
    )(table, idx)
    return out.reshape(_BATCH, _HIST, _DIM)


# speedup vs baseline: 1.8887x; 1.0077x over previous
import jax
import jax.numpy as jnp
from jax import lax
from jax.experimental import pallas as pl
from jax.experimental.pallas import tpu as pltpu
from jax.experimental.pallas import tpu_sc as plsc

_BATCH = 16384
_HIST = 50
_DIM = 64
_B = _BATCH * _HIST
_NC = 2
_NS = 16
_NW = _NC * _NS
_B_PER_W = _B // _NW
_C = 128
_NCHUNKS = _B_PER_W // _C
_NSLOT = 8
_K = 4
_NGROUPS = _NCHUNKS // _NSLOT


def _gather_body(table_hbm, idx_hbm, out_hbm, idx_all, rows, *sems):
    gsem = sems[:_NSLOT]
    ssem = sems[_NSLOT:]
    wid = lax.axis_index("s") * _NC + lax.axis_index("c")
    base = wid * _B_PER_W

    pltpu.sync_copy(idx_hbm.at[wid], idx_all)

    def start_gather(slot, chunk):
        pltpu.async_copy(table_hbm.at[idx_all.at[chunk]], rows.at[slot],
                         gsem[slot])

    def wait_gather(slot):
        pltpu.make_async_copy(table_hbm.at[idx_all.at[0]], rows.at[slot],
                              gsem[slot]).wait()

    def start_store(slot, chunk):
        pltpu.async_copy(rows.at[slot],
                         out_hbm.at[pl.ds(base + chunk * _C, _C)],
                         ssem[slot])

    def wait_store(slot):
        pltpu.make_async_copy(rows.at[slot], out_hbm.at[pl.ds(base, _C)],
                              ssem[slot]).wait()


    for t in range(_K):
        start_gather(t, t)

    for b in range(_NSLOT):
        wait_gather(b)
        start_store(b, b)
        su = (b + _K) % _NSLOT
        if b >= _NSLOT - _K:
            wait_store(su)
        start_gather(su, b + _K)

    def group(g, carry):
        for b in range(_NSLOT):
            t = g * _NSLOT + b
            wait_gather(b)
            start_store(b, t)
            su = (b + _K) % _NSLOT
            wait_store(su)
            start_gather(su, t + _K)
        return carry

    lax.fori_loop(1, _NGROUPS - 1, group, 0)

    for b in range(_NSLOT):
        t = (_NGROUPS - 1) * _NSLOT + b
        wait_gather(b)
        start_store(b, t)
        if b < _NSLOT - _K:
            su = (b + _K) % _NSLOT
            wait_store(su)
            start_gather(su, t + _K)

    for b in range(_NSLOT):
        wait_store(b)


@jax.jit
def kernel(x, table):
    idx = x.reshape(_NW, _NCHUNKS, _C)
    mesh = plsc.VectorSubcoreMesh(core_axis_name="c", subcore_axis_name="s")
    out = pl.kernel(
        _gather_body,
        mesh=mesh,
        out_type=jax.ShapeDtypeStruct((_B, _DIM), jnp.float32),
        scratch_types=[
            pltpu.VMEM((_NCHUNKS, _C), jnp.int32),
            pltpu.VMEM((_NSLOT, _C, _DIM), jnp.float32),
        ] + [pltpu.SemaphoreType.DMA] * (2 * _NSLOT),
        compiler_params=pltpu.CompilerParams(use_tc_tiling_on_sc=False),
    )(table, idx)
    return out.reshape(_BATCH, _HIST, _DIM)
